# Initial kernel scaffold; baseline (speedup 1.0000x reference)
import jax
import jax.numpy as jnp
from jax.experimental import pallas as pl


def kernel(x, edge_index, W1, W2):
    raise NotImplementedError("write your pallas kernel here")


# speedup vs baseline: 4.0026x; 4.0026x over previous
import functools

import jax
import jax.numpy as jnp
from jax import lax
from jax.experimental import pallas as pl
from jax.experimental.pallas import tpu as pltpu
from jax.experimental.pallas import tpu_sc as plsc

N = 10000
E = 320000
D = 128
K = 4

NC = 2
NS = 16
NW = NC * NS

NPAD = 10240
EPT = E // NW
EPT_PAD = 10240
EPAD = EPT_PAD * NW

C2 = 2000
CL = 128

_mesh = plsc.VectorSubcoreMesh(core_axis_name="c", subcore_axis_name="s")


@functools.partial(
    pl.kernel,
    out_type=(
        jax.ShapeDtypeStruct((EPAD,), jnp.int32),
        jax.ShapeDtypeStruct((EPAD,), jnp.int32),
        jax.ShapeDtypeStruct((NC, NPAD), jnp.float32),
    ),
    mesh=_mesh,
    scratch_types=[
        pltpu.VMEM((C2,), jnp.int32),
        pltpu.VMEM((C2,), jnp.int32),
        pltpu.VMEM((C2,), jnp.int32),
        pltpu.VMEM((EPT_PAD - EPT,), jnp.int32),
        pltpu.VMEM((CL,), jnp.int32),
        pltpu.VMEM((CL,), jnp.float32),
        pltpu.VMEM((NPAD // NS,), jnp.float32),
        pltpu.VMEM_SHARED((NPAD,), jnp.float32),
    ],
)
def _edge_prep(row_h, col_h, rowp_h, colp_h, deg_h,
               rbuf, cbuf, rpbuf, padbuf, rp128, ew128, zdbuf, degacc):
    c = lax.axis_index("c")
    s = lax.axis_index("s")
    wid = c * NS + s
    ebase = wid * EPT
    obase = wid * EPT_PAD
    nr = NPAD // NS
    r0 = s * nr

    zeros16 = jnp.zeros((16,), jnp.float32)
    nvec = jnp.full((16,), N, jnp.int32)

    def zbody(i, _):
        zdbuf[pl.ds(i * 16, 16)] = zeros16
        return 0
    lax.fori_loop(0, nr // 16, zbody, 0)
    pltpu.sync_copy(zdbuf, degacc.at[pl.ds(r0, nr)])

    for k in range(EPT // C2):
        pltpu.sync_copy(row_h.at[pl.ds(ebase + k * C2, C2)], rbuf)
        pltpu.sync_copy(col_h.at[pl.ds(ebase + k * C2, C2)], cbuf)

        def body(i, _):
            r = rbuf[pl.ds(i * 16, 16)]
            cc = cbuf[pl.ds(i * 16, 16)]
            eq = r == cc
            rpbuf[pl.ds(i * 16, 16)] = jnp.where(eq, nvec, r)
            return 0
        lax.fori_loop(0, C2 // 16, body, 0)

        pltpu.sync_copy(rpbuf, rowp_h.at[pl.ds(obase + k * C2, C2)])
        pltpu.sync_copy(cbuf, colp_h.at[pl.ds(obase + k * C2, C2)])

    def pbody(i, _):
        padbuf[pl.ds(i * 16, 16)] = nvec
        return 0
    lax.fori_loop(0, (EPT_PAD - EPT) // 16, pbody, 0)
    pltpu.sync_copy(padbuf, rowp_h.at[pl.ds(obase + EPT, EPT_PAD - EPT)])
    pltpu.sync_copy(padbuf, colp_h.at[pl.ds(obase + EPT, EPT_PAD - EPT)])
    plsc.subcore_barrier()

    def chunk(k, _):
        pltpu.sync_copy(rowp_h.at[pl.ds(obase + k * CL, CL)], rp128)

        def body(i, _):
            rp = rp128[pl.ds(i * 16, 16)]
            ew128[pl.ds(i * 16, 16)] = jnp.where(rp == nvec, 0.0, 1.0).astype(
                jnp.float32)
            return 0
        lax.fori_loop(0, CL // 16, body, 0)
        pltpu.sync_copy(ew128, degacc.at[rp128], add=True)
        return 0
    lax.fori_loop(0, EPT_PAD // CL, chunk, 0)

    plsc.subcore_barrier()
    pltpu.sync_copy(degacc.at[pl.ds(r0, nr)], deg_h.at[c, pl.ds(r0, nr)])


@functools.partial(
    pl.kernel,
    out_type=(
        jax.ShapeDtypeStruct((NPAD, D), jnp.float32),
        jax.ShapeDtypeStruct((NPAD, D), jnp.float32),
    ),
    mesh=_mesh,
    scratch_types=[
        pltpu.VMEM((CL,), jnp.int32),
        pltpu.VMEM((CL,), jnp.int32),
        pltpu.VMEM((CL, D), jnp.float32),
        pltpu.VMEM((CL, D), jnp.float32),
        pltpu.VMEM_SHARED((NPAD, D), jnp.float32),
        pltpu.SemaphoreType.DMA,
    ],
)
def _lap_sc(g_h, rowp_h, colp_h, v0_h, v1_h,
            ridx, cidx, rows, zbuf, acc, sem):
    c = lax.axis_index("c")
    s = lax.axis_index("s")
    wid = c * NS + s
    ebase = wid * EPT_PAD

    zeros16 = jnp.zeros((16,), jnp.float32)

    def zbody(i, _):
        r = i // (D // 16)
        col0 = (i % (D // 16)) * 16
        zbuf[r, pl.ds(col0, 16)] = zeros16
        return 0
    lax.fori_loop(0, CL * D // 16, zbody, 0)

    nr = NPAD // NS
    r0 = s * nr
    for j in range(nr // CL):
        pltpu.sync_copy(zbuf, acc.at[pl.ds(r0 + j * CL, CL)])
    plsc.subcore_barrier()

    def chunk(k, _):
        off = ebase + k * CL
        pltpu.sync_copy(rowp_h.at[pl.ds(off, CL)], ridx)
        pltpu.sync_copy(colp_h.at[pl.ds(off, CL)], cidx)
        pltpu.async_copy(g_h.at[ridx], rows, sem).wait()
        pltpu.sync_copy(rows, acc.at[cidx], add=True)
        return 0
    lax.fori_loop(0, EPT_PAD // CL, chunk, 0)

    plsc.subcore_barrier()

    @pl.when(c == 0)
    def _():
        pltpu.sync_copy(acc.at[pl.ds(r0, nr)], v0_h.at[pl.ds(r0, nr)])

    @pl.when(c == 1)
    def _():
        pltpu.sync_copy(acc.at[pl.ds(r0, nr)], v1_h.at[pl.ds(r0, nr)])


def _dis_body(d0_ref, d1_ref, o_ref):
    deg = d0_ref[...] + d1_ref[...]
    o_ref[...] = jnp.where(deg > 0, lax.rsqrt(deg), 0.0)


def _dis_tc(deg):
    d2 = deg.reshape(NC, NPAD // 128, 128)
    out = pl.pallas_call(
        _dis_body,
        out_shape=jax.ShapeDtypeStruct((NPAD // 128, 128), jnp.float32),
    )(d2[0], d2[1])
    return out.reshape(NPAD)


_BLK = 1024


def _scale_body(h_ref, dis_ref, o_ref):
    o_ref[...] = h_ref[...] * dis_ref[...]


def _scale_tc(h, dis_col):
    grid = (NPAD // _BLK,)
    return pl.pallas_call(
        _scale_body,
        grid=grid,
        in_specs=[
            pl.BlockSpec((_BLK, D), lambda i: (i, 0)),
            pl.BlockSpec((_BLK, 1), lambda i: (i, 0)),
        ],
        out_specs=pl.BlockSpec((_BLK, D), lambda i: (i, 0)),
        out_shape=jax.ShapeDtypeStruct((NPAD, D), jnp.float32),
    )(h, dis_col)


def _combine_body(alpha, beta, y0_ref, y1_ref, dis_ref, tp_ref, tx_ref, g_ref):
    dis = dis_ref[...]
    tx = (alpha * dis) * (y0_ref[...] + y1_ref[...]) + beta * tp_ref[...]
    tx_ref[...] = tx
    g_ref[...] = dis * tx


def _combine_tc(y0, y1, dis_col, tprev, alpha, beta):
    grid = (NPAD // _BLK,)
    return pl.pallas_call(
        functools.partial(_combine_body, alpha, beta),
        grid=grid,
        in_specs=[
            pl.BlockSpec((_BLK, D), lambda i: (i, 0)),
            pl.BlockSpec((_BLK, D), lambda i: (i, 0)),
            pl.BlockSpec((_BLK, 1), lambda i: (i, 0)),
            pl.BlockSpec((_BLK, D), lambda i: (i, 0)),
        ],
        out_specs=[
            pl.BlockSpec((_BLK, D), lambda i: (i, 0)),
            pl.BlockSpec((_BLK, D), lambda i: (i, 0)),
        ],
        out_shape=[
            jax.ShapeDtypeStruct((NPAD, D), jnp.float32),
            jax.ShapeDtypeStruct((NPAD, D), jnp.float32),
        ],
    )(y0, y1, dis_col, tprev)


def _matmul_body(relu, t0_ref, t1_ref, t2_ref, t3_ref, w_ref, o_ref):
    acc = jnp.dot(t0_ref[...], w_ref[0], preferred_element_type=jnp.float32)
    acc += jnp.dot(t1_ref[...], w_ref[1], preferred_element_type=jnp.float32)
    acc += jnp.dot(t2_ref[...], w_ref[2], preferred_element_type=jnp.float32)
    acc += jnp.dot(t3_ref[...], w_ref[3], preferred_element_type=jnp.float32)
    if relu:
        acc = jnp.maximum(acc, 0.0)
    o_ref[...] = acc


def _matmul_tc(t0, t1, t2, t3, w, relu):
    grid = (NPAD // _BLK,)
    bs = pl.BlockSpec((_BLK, D), lambda i: (i, 0))
    return pl.pallas_call(
        functools.partial(_matmul_body, relu),
        grid=grid,
        in_specs=[bs, bs, bs, bs,
                  pl.BlockSpec((K, D, D), lambda i: (0, 0, 0))],
        out_specs=bs,
        out_shape=jax.ShapeDtypeStruct((NPAD, D), jnp.float32),
    )(t0, t1, t2, t3, w)


def _layer(h, rowp, colp, dis_col, w, relu):
    tx0 = h
    g = _scale_tc(tx0, dis_col)
    y0, y1 = _lap_sc(g, rowp, colp)
    tx1, g = _combine_tc(y0, y1, dis_col, tx0, -1.0, 0.0)
    y0, y1 = _lap_sc(g, rowp, colp)
    tx2, g = _combine_tc(y0, y1, dis_col, tx0, -2.0, -1.0)
    y0, y1 = _lap_sc(g, rowp, colp)
    tx3, _ = _combine_tc(y0, y1, dis_col, tx1, -2.0, -1.0)
    return _matmul_tc(tx0, tx1, tx2, tx3, w, relu)


def kernel(x, edge_index, W1, W2):
    row = edge_index[0]
    col = edge_index[1]
    rowp, colp, deg = _edge_prep(row, col)
    dis = _dis_tc(deg)
    dis_col = dis.reshape(NPAD, 1)

    xpad = jnp.pad(x, ((0, NPAD - N), (0, 0)))
    w1 = W1
    w2 = jnp.pad(W2, ((0, 0), (0, 0), (0, D - W2.shape[2])))

    h = _layer(xpad, rowp, colp, dis_col, w1, True)
    out = _layer(h, rowp, colp, dis_col, w2, False)
    return out[:N, : W2.shape[2]]


# speedup vs baseline: 5.2544x; 1.3127x over previous
import functools

import jax
import jax.numpy as jnp
from jax import lax
from jax.experimental import pallas as pl
from jax.experimental.pallas import tpu as pltpu
from jax.experimental.pallas import tpu_sc as plsc

N = 10000
E = 320000
D = 128
K = 4

NC = 2
NS = 16
NW = NC * NS

NPAD = 10240
EPT = E // NW
EPT_PAD = 10240
EPAD = EPT_PAD * NW

C2 = 2000
CP = 128
CL = 64
NCH = EPT_PAD // CL
NB = 3
NIB = 16
NBI = NCH // NIB

_mesh = plsc.VectorSubcoreMesh(core_axis_name="c", subcore_axis_name="s")


@functools.partial(
    pl.kernel,
    out_type=(
        jax.ShapeDtypeStruct((EPAD,), jnp.int32),
        jax.ShapeDtypeStruct((EPAD,), jnp.int32),
        jax.ShapeDtypeStruct((NC, NPAD), jnp.float32),
    ),
    mesh=_mesh,
    scratch_types=[
        pltpu.VMEM((C2,), jnp.int32),
        pltpu.VMEM((C2,), jnp.int32),
        pltpu.VMEM((C2,), jnp.int32),
        pltpu.VMEM((EPT_PAD - EPT,), jnp.int32),
        pltpu.VMEM((CP,), jnp.int32),
        pltpu.VMEM((CP,), jnp.float32),
        pltpu.VMEM((NPAD // NS,), jnp.float32),
        pltpu.VMEM_SHARED((NPAD,), jnp.float32),
    ],
)
def _edge_prep(row_h, col_h, rowp_h, colp_h, deg_h,
               rbuf, cbuf, rpbuf, padbuf, rp128, ew128, zdbuf, degacc):
    c = lax.axis_index("c")
    s = lax.axis_index("s")
    wid = c * NS + s
    ebase = wid * EPT
    obase = wid * EPT_PAD
    nr = NPAD // NS
    r0 = s * nr

    zeros16 = jnp.zeros((16,), jnp.float32)
    nvec = jnp.full((16,), N, jnp.int32)

    def zbody(i, _):
        zdbuf[pl.ds(i * 16, 16)] = zeros16
        return 0
    lax.fori_loop(0, nr // 16, zbody, 0)
    pltpu.sync_copy(zdbuf, degacc.at[pl.ds(r0, nr)])

    for k in range(EPT // C2):
        pltpu.sync_copy(row_h.at[pl.ds(ebase + k * C2, C2)], rbuf)
        pltpu.sync_copy(col_h.at[pl.ds(ebase + k * C2, C2)], cbuf)

        def body(i, _):
            r = rbuf[pl.ds(i * 16, 16)]
            cc = cbuf[pl.ds(i * 16, 16)]
            eq = r == cc
            rpbuf[pl.ds(i * 16, 16)] = jnp.where(eq, nvec, r)
            return 0
        lax.fori_loop(0, C2 // 16, body, 0)

        pltpu.sync_copy(rpbuf, rowp_h.at[pl.ds(obase + k * C2, C2)])
        pltpu.sync_copy(cbuf, colp_h.at[pl.ds(obase + k * C2, C2)])

    def pbody(i, _):
        padbuf[pl.ds(i * 16, 16)] = nvec
        return 0
    lax.fori_loop(0, (EPT_PAD - EPT) // 16, pbody, 0)
    pltpu.sync_copy(padbuf, rowp_h.at[pl.ds(obase + EPT, EPT_PAD - EPT)])
    pltpu.sync_copy(padbuf, colp_h.at[pl.ds(obase + EPT, EPT_PAD - EPT)])
    plsc.subcore_barrier()

    def chunk(k, _):
        pltpu.sync_copy(rowp_h.at[pl.ds(obase + k * CP, CP)], rp128)

        def body(i, _):
            rp = rp128[pl.ds(i * 16, 16)]
            ew128[pl.ds(i * 16, 16)] = jnp.where(rp == nvec, 0.0, 1.0).astype(
                jnp.float32)
            return 0
        lax.fori_loop(0, CP // 16, body, 0)
        pltpu.sync_copy(ew128, degacc.at[rp128], add=True)
        return 0
    lax.fori_loop(0, EPT_PAD // CP, chunk, 0)

    plsc.subcore_barrier()
    pltpu.sync_copy(degacc.at[pl.ds(r0, nr)], deg_h.at[c, pl.ds(r0, nr)])


@functools.partial(
    pl.kernel,
    out_type=(
        jax.ShapeDtypeStruct((NPAD, D), jnp.float32),
        jax.ShapeDtypeStruct((NPAD, D), jnp.float32),
    ),
    mesh=_mesh,
    scratch_types=[
        [pltpu.VMEM((NIB, CL), jnp.int32) for _ in range(2)],
        [pltpu.VMEM((NIB, CL), jnp.int32) for _ in range(2)],
        [pltpu.VMEM((CL, D), jnp.float32) for _ in range(NB)],
        [pltpu.SemaphoreType.DMA for _ in range(2)],
        [pltpu.SemaphoreType.DMA for _ in range(2)],
        [pltpu.SemaphoreType.DMA for _ in range(NB)],
        [pltpu.SemaphoreType.DMA for _ in range(NB)],
        pltpu.VMEM_SHARED((NPAD, D), jnp.float32),
    ],
)
def _lap_sc(g_h, rowp3_h, colp3_h, v0_h, v1_h,
            ridxb, cidxb, rows, risem, cisem, gsem, ssem, acc):
    c = lax.axis_index("c")
    s = lax.axis_index("s")
    wid = c * NS + s
    ibase = wid * NCH

    zeros16 = jnp.zeros((16,), jnp.float32)

    def zbody(i, _):
        r = i // (D // 16)
        col0 = (i % (D // 16)) * 16
        rows[0][r, pl.ds(col0, 16)] = zeros16
        return 0
    lax.fori_loop(0, CL * D // 16, zbody, 0)

    nr = NPAD // NS
    r0 = s * nr
    for j in range(nr // CL):
        pltpu.sync_copy(rows[0], acc.at[pl.ds(r0 + j * CL, CL)])
    plsc.subcore_barrier()

    pltpu.sync_copy(rowp3_h.at[pl.ds(ibase, NIB)], ridxb[0])
    pltpu.sync_copy(colp3_h.at[pl.ds(ibase, NIB)], cidxb[0])

    gd = [None] * NCH
    sd = [None] * NCH
    rd = [None] * NBI
    cd = [None] * NBI

    def scatter(kk):
        bb = kk % NB
        mi, ji = divmod(kk, NIB)
        gd[kk].wait()
        sd[kk] = pltpu.async_copy(
            rows[bb], acc.at[cidxb[mi % 2].at[ji]], ssem[bb], add=True)

    for k in range(NCH):
        m, j = divmod(k, NIB)
        if k >= NB:
            sd[k - NB].wait()
        if j == 2 and m + 1 < NBI:
            nbuf = (m + 1) % 2
            rd[m + 1] = pltpu.async_copy(
                rowp3_h.at[pl.ds(ibase + (m + 1) * NIB, NIB)],
                ridxb[nbuf], risem[nbuf])
            cd[m + 1] = pltpu.async_copy(
                colp3_h.at[pl.ds(ibase + (m + 1) * NIB, NIB)],
                cidxb[nbuf], cisem[nbuf])
        if j == 0 and m > 0:
            rd[m].wait()
            cd[m].wait()
        b = k % NB
        gd[k] = pltpu.async_copy(g_h.at[ridxb[m % 2].at[j]], rows[b], gsem[b])
        if k >= 2:
            scatter(k - 2)
    for kk in range(NCH - 2, NCH):
        scatter(kk)
    for kk in range(NCH - NB, NCH):
        sd[kk].wait()

    plsc.subcore_barrier()

    @pl.when(c == 0)
    def _():
        pltpu.sync_copy(acc.at[pl.ds(r0, nr)], v0_h.at[pl.ds(r0, nr)])

    @pl.when(c == 1)
    def _():
        pltpu.sync_copy(acc.at[pl.ds(r0, nr)], v1_h.at[pl.ds(r0, nr)])


def _dis_body(d0_ref, d1_ref, o_ref):
    deg = d0_ref[...] + d1_ref[...]
    o_ref[...] = jnp.where(deg > 0, lax.rsqrt(deg), 0.0)


def _dis_tc(deg):
    d2 = deg.reshape(NC, NPAD // 128, 128)
    out = pl.pallas_call(
        _dis_body,
        out_shape=jax.ShapeDtypeStruct((NPAD // 128, 128), jnp.float32),
    )(d2[0], d2[1])
    return out.reshape(NPAD)


_BLK = 1024
_NBLK = NPAD // _BLK


def _scale_body(h_ref, dis_ref, o_ref):
    o_ref[...] = h_ref[...] * dis_ref[...]


def _scale_tc(h, dis_col):
    grid = (_NBLK,)
    return pl.pallas_call(
        _scale_body,
        grid=grid,
        in_specs=[
            pl.BlockSpec((_BLK, D), lambda i: (i, 0)),
            pl.BlockSpec((_BLK, 1), lambda i: (i, 0)),
        ],
        out_specs=pl.BlockSpec((_BLK, D), lambda i: (i, 0)),
        out_shape=jax.ShapeDtypeStruct((NPAD, D), jnp.float32),
    )(h, dis_col)


def _combine_body(alpha, beta, y0_ref, y1_ref, dis_ref, tp_ref, tx_ref, g_ref):
    dis = dis_ref[...]
    tx = (alpha * dis) * (y0_ref[...] + y1_ref[...]) + beta * tp_ref[...]
    tx_ref[...] = tx
    g_ref[...] = dis * tx


def _combine_tc(y0, y1, dis_col, tprev, alpha, beta):
    grid = (_NBLK,)
    bs = pl.BlockSpec((_BLK, D), lambda i: (i, 0))
    return pl.pallas_call(
        functools.partial(_combine_body, alpha, beta),
        grid=grid,
        in_specs=[bs, bs, pl.BlockSpec((_BLK, 1), lambda i: (i, 0)), bs],
        out_specs=[bs, bs],
        out_shape=[
            jax.ShapeDtypeStruct((NPAD, D), jnp.float32),
            jax.ShapeDtypeStruct((NPAD, D), jnp.float32),
        ],
    )(y0, y1, dis_col, tprev)


def _matmul_body(relu, t0_ref, t1_ref, t2_ref, t3_ref, w_ref, o_ref):
    acc = jnp.dot(t0_ref[...], w_ref[0], preferred_element_type=jnp.float32)
    acc += jnp.dot(t1_ref[...], w_ref[1], preferred_element_type=jnp.float32)
    acc += jnp.dot(t2_ref[...], w_ref[2], preferred_element_type=jnp.float32)
    acc += jnp.dot(t3_ref[...], w_ref[3], preferred_element_type=jnp.float32)
    if relu:
        acc = jnp.maximum(acc, 0.0)
    o_ref[...] = acc


def _matmul_tc(t0, t1, t2, t3, w, relu):
    grid = (_NBLK,)
    bs = pl.BlockSpec((_BLK, D), lambda i: (i, 0))
    return pl.pallas_call(
        functools.partial(_matmul_body, relu),
        grid=grid,
        in_specs=[bs, bs, bs, bs,
                  pl.BlockSpec((K, D, D), lambda i: (0, 0, 0))],
        out_specs=bs,
        out_shape=jax.ShapeDtypeStruct((NPAD, D), jnp.float32),
    )(t0, t1, t2, t3, w)


def _layer(h, rowp, colp, dis_col, w, relu):
    tx0 = h
    g = _scale_tc(tx0, dis_col)
    y0, y1 = _lap_sc(g, rowp, colp)
    tx1, g = _combine_tc(y0, y1, dis_col, tx0, -1.0, 0.0)
    y0, y1 = _lap_sc(g, rowp, colp)
    tx2, g = _combine_tc(y0, y1, dis_col, tx0, -2.0, -1.0)
    y0, y1 = _lap_sc(g, rowp, colp)
    tx3, _ = _combine_tc(y0, y1, dis_col, tx1, -2.0, -1.0)
    return _matmul_tc(tx0, tx1, tx2, tx3, w, relu)


def kernel(x, edge_index, W1, W2):
    row = edge_index[0]
    col = edge_index[1]
    rowp, colp, deg = _edge_prep(row, col)
    rowp = rowp.reshape(NW * NCH, CL)
    colp = colp.reshape(NW * NCH, CL)
    dis = _dis_tc(deg)
    dis_col = dis.reshape(NPAD, 1)

    xpad = jnp.pad(x, ((0, NPAD - N), (0, 0)))
    w1 = W1
    w2 = jnp.pad(W2, ((0, 0), (0, 0), (0, D - W2.shape[2])))

    h = _layer(xpad, rowp, colp, dis_col, w1, True)
    out = _layer(h, rowp, colp, dis_col, w2, False)
    return out[:N, : W2.shape[2]]


# speedup vs baseline: 5.3184x; 1.0122x over previous
import functools

import jax
import jax.numpy as jnp
from jax import lax
from jax.experimental import pallas as pl
from jax.experimental.pallas import tpu as pltpu
from jax.experimental.pallas import tpu_sc as plsc

N = 10000
E = 320000
D = 128
K = 4

NC = 2
NS = 16
NW = NC * NS

NPAD = 10240
EPT = E // NW
EPT_PAD = 10240
EPAD = EPT_PAD * NW

C2 = 2000
CP = 128
CL = 32
NCH = EPT_PAD // CL
NB = 6
GLAG = 4
NIB = 32
NBI = NCH // NIB

_mesh = plsc.VectorSubcoreMesh(core_axis_name="c", subcore_axis_name="s")


@functools.partial(
    pl.kernel,
    out_type=(
        jax.ShapeDtypeStruct((EPAD,), jnp.int32),
        jax.ShapeDtypeStruct((EPAD,), jnp.int32),
        jax.ShapeDtypeStruct((NC, NPAD), jnp.float32),
    ),
    mesh=_mesh,
    scratch_types=[
        pltpu.VMEM((C2,), jnp.int32),
        pltpu.VMEM((C2,), jnp.int32),
        pltpu.VMEM((EPT_PAD,), jnp.int32),
        [pltpu.VMEM((CP,), jnp.int32) for _ in range(2)],
        [pltpu.VMEM((CP,), jnp.float32) for _ in range(2)],
        [pltpu.SemaphoreType.DMA for _ in range(2)],
        pltpu.VMEM((NPAD // NS,), jnp.float32),
        pltpu.VMEM_SHARED((NPAD,), jnp.float32),
    ],
)
def _edge_prep(row_h, col_h, rowp_h, colp_h, deg_h,
               rbuf, cbuf, rc, rpb, ewb, dsem, zdbuf, degacc):
    c = lax.axis_index("c")
    s = lax.axis_index("s")
    wid = c * NS + s
    ebase = wid * EPT
    obase = wid * EPT_PAD
    nr = NPAD // NS
    r0 = s * nr

    zeros16 = jnp.zeros((16,), jnp.float32)
    nvec = jnp.full((16,), N, jnp.int32)

    def zbody(i, _):
        zdbuf[pl.ds(i * 16, 16)] = zeros16
        return 0
    lax.fori_loop(0, nr // 16, zbody, 0)
    pltpu.sync_copy(zdbuf, degacc.at[pl.ds(r0, nr)])

    for k in range(EPT // C2):
        pltpu.sync_copy(row_h.at[pl.ds(ebase + k * C2, C2)], rbuf)
        pltpu.sync_copy(col_h.at[pl.ds(ebase + k * C2, C2)], cbuf)

        def body(i, _):
            r = rbuf[pl.ds(i * 16, 16)]
            cc = cbuf[pl.ds(i * 16, 16)]
            eq = r == cc
            rc[pl.ds(k * C2 + i * 16, 16)] = jnp.where(eq, nvec, r)
            return 0
        lax.fori_loop(0, C2 // 16, body, 0)

        pltpu.sync_copy(cbuf, colp_h.at[pl.ds(obase + k * C2, C2)])

    def pbody(i, _):
        rc[pl.ds(EPT + i * 16, 16)] = nvec
        return 0
    lax.fori_loop(0, (EPT_PAD - EPT) // 16, pbody, 0)
    pltpu.sync_copy(rc, rowp_h.at[pl.ds(obase, EPT_PAD)])
    pltpu.sync_copy(rc.at[pl.ds(EPT, EPT_PAD - EPT)],
                    colp_h.at[pl.ds(obase + EPT, EPT_PAD - EPT)])
    plsc.subcore_barrier()

    sd = [None] * (EPT_PAD // CP)
    for k in range(EPT_PAD // CP):
        b = k % 2
        if k >= 2:
            sd[k - 2].wait()

        def body(i, _):
            rp = rc[pl.ds(k * CP + i * 16, 16)]
            rpb[b][pl.ds(i * 16, 16)] = rp
            ewb[b][pl.ds(i * 16, 16)] = jnp.where(rp == nvec, 0.0, 1.0).astype(
                jnp.float32)
            return 0
        lax.fori_loop(0, CP // 16, body, 0)
        sd[k] = pltpu.async_copy(ewb[b], degacc.at[rpb[b]], dsem[b], add=True)
    sd[-2].wait()
    sd[-1].wait()

    plsc.subcore_barrier()
    pltpu.sync_copy(degacc.at[pl.ds(r0, nr)], deg_h.at[c, pl.ds(r0, nr)])


@functools.partial(
    pl.kernel,
    out_type=(
        jax.ShapeDtypeStruct((NPAD, D), jnp.float32),
        jax.ShapeDtypeStruct((NPAD, D), jnp.float32),
    ),
    mesh=_mesh,
    scratch_types=[
        [pltpu.VMEM((NIB, CL), jnp.int32) for _ in range(2)],
        [pltpu.VMEM((NIB, CL), jnp.int32) for _ in range(2)],
        [pltpu.VMEM((CL, D), jnp.float32) for _ in range(NB)],
        [pltpu.SemaphoreType.DMA for _ in range(2)],
        [pltpu.SemaphoreType.DMA for _ in range(2)],
        [pltpu.SemaphoreType.DMA for _ in range(NB)],
        [pltpu.SemaphoreType.DMA for _ in range(NB)],
        pltpu.VMEM_SHARED((NPAD, D), jnp.float32),
    ],
)
def _lap_sc(g_h, rowp3_h, colp3_h, v0_h, v1_h,
            ridxb, cidxb, rows, risem, cisem, gsem, ssem, acc):
    c = lax.axis_index("c")
    s = lax.axis_index("s")
    wid = c * NS + s
    ibase = wid * NCH

    zeros16 = jnp.zeros((16,), jnp.float32)

    def zbody(i, _):
        r = i // (D // 16)
        col0 = (i % (D // 16)) * 16
        rows[0][r, pl.ds(col0, 16)] = zeros16
        return 0
    lax.fori_loop(0, CL * D // 16, zbody, 0)

    nr = NPAD // NS
    r0 = s * nr
    for j in range(nr // CL):
        pltpu.sync_copy(rows[0], acc.at[pl.ds(r0 + j * CL, CL)])
    plsc.subcore_barrier()

    pltpu.sync_copy(rowp3_h.at[pl.ds(ibase, NIB)], ridxb[0])
    pltpu.sync_copy(colp3_h.at[pl.ds(ibase, NIB)], cidxb[0])

    gd = [None] * NCH
    sd = [None] * NCH
    rd = [None] * NBI
    cd = [None] * NBI

    def scatter(kk):
        bb = kk % NB
        mi, ji = divmod(kk, NIB)
        gd[kk].wait()
        sd[kk] = pltpu.async_copy(
            rows[bb], acc.at[cidxb[mi % 2].at[ji]], ssem[bb], add=True)

    for k in range(NCH):
        m, j = divmod(k, NIB)
        if k >= NB:
            sd[k - NB].wait()
        if j == NB and m + 1 < NBI:
            nbuf = (m + 1) % 2
            rd[m + 1] = pltpu.async_copy(
                rowp3_h.at[pl.ds(ibase + (m + 1) * NIB, NIB)],
                ridxb[nbuf], risem[nbuf])
            cd[m + 1] = pltpu.async_copy(
                colp3_h.at[pl.ds(ibase + (m + 1) * NIB, NIB)],
                cidxb[nbuf], cisem[nbuf])
        if j == 0 and m > 0:
            rd[m].wait()
            cd[m].wait()
        b = k % NB
        gd[k] = pltpu.async_copy(g_h.at[ridxb[m % 2].at[j]], rows[b], gsem[b])
        if k >= GLAG:
            scatter(k - GLAG)
    for kk in range(NCH - GLAG, NCH):
        scatter(kk)
    for kk in range(NCH - NB, NCH):
        sd[kk].wait()

    plsc.subcore_barrier()

    @pl.when(c == 0)
    def _():
        pltpu.sync_copy(acc.at[pl.ds(r0, nr)], v0_h.at[pl.ds(r0, nr)])

    @pl.when(c == 1)
    def _():
        pltpu.sync_copy(acc.at[pl.ds(r0, nr)], v1_h.at[pl.ds(r0, nr)])


def _dis_body(d0_ref, d1_ref, o_ref):
    deg = d0_ref[...] + d1_ref[...]
    o_ref[...] = jnp.where(deg > 0, lax.rsqrt(deg), 0.0)


def _dis_tc(deg):
    d2 = deg.reshape(NC, NPAD // 128, 128)
    out = pl.pallas_call(
        _dis_body,
        out_shape=jax.ShapeDtypeStruct((NPAD // 128, 128), jnp.float32),
    )(d2[0], d2[1])
    return out.reshape(NPAD)


_BLK = 1024
_NBLK = NPAD // _BLK


def _scale_body(h_ref, dis_ref, o_ref):
    o_ref[...] = h_ref[...] * dis_ref[...]


def _scale_tc(h, dis_col):
    grid = (_NBLK,)
    return pl.pallas_call(
        _scale_body,
        grid=grid,
        in_specs=[
            pl.BlockSpec((_BLK, D), lambda i: (i, 0)),
            pl.BlockSpec((_BLK, 1), lambda i: (i, 0)),
        ],
        out_specs=pl.BlockSpec((_BLK, D), lambda i: (i, 0)),
        out_shape=jax.ShapeDtypeStruct((NPAD, D), jnp.float32),
    )(h, dis_col)


def _combine_body(alpha, beta, y0_ref, y1_ref, dis_ref, tp_ref, tx_ref, g_ref):
    dis = dis_ref[...]
    tx = (alpha * dis) * (y0_ref[...] + y1_ref[...]) + beta * tp_ref[...]
    tx_ref[...] = tx
    g_ref[...] = dis * tx


def _combine_tc(y0, y1, dis_col, tprev, alpha, beta):
    grid = (_NBLK,)
    bs = pl.BlockSpec((_BLK, D), lambda i: (i, 0))
    return pl.pallas_call(
        functools.partial(_combine_body, alpha, beta),
        grid=grid,
        in_specs=[bs, bs, pl.BlockSpec((_BLK, 1), lambda i: (i, 0)), bs],
        out_specs=[bs, bs],
        out_shape=[
            jax.ShapeDtypeStruct((NPAD, D), jnp.float32),
            jax.ShapeDtypeStruct((NPAD, D), jnp.float32),
        ],
    )(y0, y1, dis_col, tprev)


def _matmul_body(relu, t0_ref, t1_ref, t2_ref, t3_ref, w_ref, o_ref):
    acc = jnp.dot(t0_ref[...], w_ref[0], preferred_element_type=jnp.float32)
    acc += jnp.dot(t1_ref[...], w_ref[1], preferred_element_type=jnp.float32)
    acc += jnp.dot(t2_ref[...], w_ref[2], preferred_element_type=jnp.float32)
    acc += jnp.dot(t3_ref[...], w_ref[3], preferred_element_type=jnp.float32)
    if relu:
        acc = jnp.maximum(acc, 0.0)
    o_ref[...] = acc


def _matmul_tc(t0, t1, t2, t3, w, relu):
    grid = (_NBLK,)
    bs = pl.BlockSpec((_BLK, D), lambda i: (i, 0))
    return pl.pallas_call(
        functools.partial(_matmul_body, relu),
        grid=grid,
        in_specs=[bs, bs, bs, bs,
                  pl.BlockSpec((K, D, D), lambda i: (0, 0, 0))],
        out_specs=bs,
        out_shape=jax.ShapeDtypeStruct((NPAD, D), jnp.float32),
    )(t0, t1, t2, t3, w)


def _layer(h, rowp, colp, dis_col, w, relu):
    tx0 = h
    g = _scale_tc(tx0, dis_col)
    y0, y1 = _lap_sc(g, rowp, colp)
    tx1, g = _combine_tc(y0, y1, dis_col, tx0, -1.0, 0.0)
    y0, y1 = _lap_sc(g, rowp, colp)
    tx2, g = _combine_tc(y0, y1, dis_col, tx0, -2.0, -1.0)
    y0, y1 = _lap_sc(g, rowp, colp)
    tx3, _ = _combine_tc(y0, y1, dis_col, tx1, -2.0, -1.0)
    return _matmul_tc(tx0, tx1, tx2, tx3, w, relu)


def kernel(x, edge_index, W1, W2):
    row = edge_index[0]
    col = edge_index[1]
    rowp, colp, deg = _edge_prep(row, col)
    rowp = rowp.reshape(NW * NCH, CL)
    colp = colp.reshape(NW * NCH, CL)
    dis = _dis_tc(deg)
    dis_col = dis.reshape(NPAD, 1)

    xpad = jnp.pad(x, ((0, NPAD - N), (0, 0)))
    w1 = W1
    w2 = jnp.pad(W2, ((0, 0), (0, 0), (0, D - W2.shape[2])))

    h = _layer(xpad, rowp, colp, dis_col, w1, True)
    out = _layer(h, rowp, colp, dis_col, w2, False)
    return out[:N, : W2.shape[2]]
